# Initial kernel scaffold; baseline (speedup 1.0000x reference)
import jax
import jax.numpy as jnp
from jax.experimental import pallas as pl


def kernel(x, edge_index, W_gcn, b_gcn, W1, b1, W2, b2, Wc, bc):
    raise NotImplementedError("write your pallas kernel here")


# speedup vs baseline: 72.7217x; 72.7217x over previous
import functools

import jax
import jax.numpy as jnp
from jax import lax
from jax.experimental import pallas as pl
from jax.experimental.pallas import tpu as pltpu
from jax.experimental.pallas import tpu_sc as plsc

_NS = 16
_B = 128
_L = 16


def _rsqrt16(v):
    t = 0.5 * (v + 1.0)
    for _ in range(13):
        t = 0.5 * (t + v / t)
    return 1.0 / t


def _make_sc_w(rows, npad):
    sl = npad // _NS
    nk = sl // _L
    mesh = plsc.VectorSubcoreMesh(
        core_axis_name="c", subcore_axis_name="s", num_cores=1)

    @functools.partial(
        pl.kernel,
        mesh=mesh,
        out_type=jax.ShapeDtypeStruct((npad,), jnp.float32),
        scratch_types=[
            pltpu.VMEM((rows, _B), jnp.int32),
            pltpu.VMEM((rows, _B), jnp.int32),
            pltpu.VMEM((_B,), jnp.float32),
            pltpu.VMEM((sl,), jnp.float32),
            pltpu.VMEM((sl,), jnp.float32),
            pltpu.VMEM_SHARED((npad,), jnp.float32),
            pltpu.VMEM_SHARED((npad,), jnp.float32),
            pltpu.VMEM_SHARED((npad,), jnp.float32),
        ],
    )
    def sc_w(src_hbm, dst_hbm, w_hbm, src_v, dst_v, valr, dsl, ssl,
             sh_deg, sh_d, sh_s):
        wid = lax.axis_index("s")
        base = wid * sl

        pltpu.sync_copy(src_hbm.at[wid], src_v)
        pltpu.sync_copy(dst_hbm.at[wid], dst_v)

        for k in range(nk):
            dsl[pl.ds(k * _L, _L)] = jnp.zeros((_L,), jnp.float32)
        pltpu.sync_copy(dsl, sh_deg.at[pl.ds(base, sl)])
        pltpu.sync_copy(dsl, sh_s.at[pl.ds(base, sl)])
        for k in range(_B // _L):
            valr[pl.ds(k * _L, _L)] = jnp.ones((_L,), jnp.float32)
        plsc.subcore_barrier()

        def p1(j, carry):
            pltpu.sync_copy(valr, sh_deg.at[dst_v.at[j]], add=True)
            return carry
        lax.fori_loop(0, rows, p1, 0)
        plsc.subcore_barrier()

        pltpu.sync_copy(sh_deg.at[pl.ds(base, sl)], dsl)
        for k in range(nk):
            v = dsl[pl.ds(k * _L, _L)] + 1.0
            dsl[pl.ds(k * _L, _L)] = _rsqrt16(v)
        pltpu.sync_copy(dsl, sh_d.at[pl.ds(base, sl)])
        plsc.subcore_barrier()

        def p3(j, carry):
            pltpu.sync_copy(sh_d.at[dst_v.at[j]], valr)
            pltpu.sync_copy(valr, sh_s.at[src_v.at[j]], add=True)
            return carry
        lax.fori_loop(0, rows, p3, 0)
        plsc.subcore_barrier()

        pltpu.sync_copy(sh_s.at[pl.ds(base, sl)], ssl)
        for k in range(nk):
            d = dsl[pl.ds(k * _L, _L)]
            s = ssl[pl.ds(k * _L, _L)]
            ssl[pl.ds(k * _L, _L)] = d * s + d * d
        pltpu.sync_copy(ssl, w_hbm.at[pl.ds(base, sl)])

    return sc_w


def _tc_head(nfloat):
    def body(w_ref, x_ref, wg_ref, bg_ref, w1_ref, b1_ref, w2_ref, b2_ref,
             wc_ref, bc_ref, out_ref):
        xw = jnp.dot(w_ref[...], x_ref[...],
                     preferred_element_type=jnp.float32)
        pooled = jnp.dot(xw, wg_ref[...],
                         preferred_element_type=jnp.float32)
        pooled = pooled + nfloat * bg_ref[...]
        h = jnp.maximum(jnp.dot(pooled, w1_ref[...],
                                preferred_element_type=jnp.float32)
                        + b1_ref[...], 0.0)
        h = jnp.maximum(jnp.dot(h, w2_ref[...],
                                preferred_element_type=jnp.float32)
                        + b2_ref[...], 0.0)
        avg = jnp.dot(h, wc_ref[...],
                      preferred_element_type=jnp.float32) + bc_ref[...]
        m = jnp.max(avg, axis=1, keepdims=True)
        e = jnp.exp(avg - m)
        out_ref[...] = e / jnp.sum(e, axis=1, keepdims=True)
    return body


@jax.jit
def kernel(x, edge_index, W_gcn, b_gcn, W1, b1, W2, b2, Wc, bc):
    n, _ = x.shape
    e = edge_index.shape[1]

    rows = -(-e // (_NS * _B))
    epad = _NS * rows * _B
    npad = -(-(n + 1) // (_NS * _L)) * (_NS * _L)

    fill = jnp.full((epad - e,), n, jnp.int32)
    src3 = jnp.concatenate([edge_index[0], fill]).reshape(_NS, rows, _B)
    dst3 = jnp.concatenate([edge_index[1], fill]).reshape(_NS, rows, _B)

    w_full = _make_sc_w(rows, npad)(src3, dst3)
    w_row = w_full[:n].reshape(1, n)

    result = pl.pallas_call(
        _tc_head(float(n)),
        out_shape=jax.ShapeDtypeStruct((1, 2), jnp.float32),
    )(w_row, x, W_gcn, b_gcn.reshape(1, -1), W1, b1.reshape(1, -1),
      W2, b2.reshape(1, -1), Wc, bc.reshape(1, -1))
    return (result, x)


# speedup vs baseline: 86.8635x; 1.1945x over previous
import functools

import jax
import jax.numpy as jnp
from jax import lax
from jax.experimental import pallas as pl
from jax.experimental.pallas import tpu as pltpu
from jax.experimental.pallas import tpu_sc as plsc

_NS = 16
_B = 128
_L = 16


def _rsqrt16(v):
    t = 0.5 * (v + 1.0)
    for _ in range(13):
        t = 0.5 * (t + v / t)
    return 1.0 / t


def _make_sc_w(rows, npad):
    sl = npad // _NS
    nk = sl // _L
    mesh = plsc.VectorSubcoreMesh(
        core_axis_name="c", subcore_axis_name="s", num_cores=1)

    grp = 26

    def _fire_drain(mk):
        bounds = list(range(0, rows, grp)) + [rows]
        prev = [mk(j) for j in range(bounds[0], bounds[1])]
        for gi in range(1, len(bounds) - 1):
            cur = [mk(j) for j in range(bounds[gi], bounds[gi + 1])]
            for c in prev:
                c.wait()
            prev = cur
        for c in prev:
            c.wait()

    @functools.partial(
        pl.kernel,
        mesh=mesh,
        out_type=jax.ShapeDtypeStruct((npad,), jnp.float32),
        scratch_types=[
            pltpu.VMEM((rows, _B), jnp.int32),
            pltpu.VMEM((rows, _B), jnp.int32),
            pltpu.VMEM((_B,), jnp.float32),
            pltpu.VMEM((rows, _B), jnp.float32),
            pltpu.VMEM((sl,), jnp.float32),
            pltpu.VMEM((sl,), jnp.float32),
            pltpu.VMEM_SHARED((npad,), jnp.float32),
            pltpu.VMEM_SHARED((npad,), jnp.float32),
            pltpu.VMEM_SHARED((npad,), jnp.float32),
            pltpu.SemaphoreType.DMA,
        ],
    )
    def sc_w(src_hbm, dst_hbm, w_hbm, src_v, dst_v, valr, vals, dsl, ssl,
             sh_deg, sh_d, sh_s, sem):
        wid = lax.axis_index("s")
        base = wid * sl

        cin = [pltpu.async_copy(src_hbm.at[wid], src_v, sem),
               pltpu.async_copy(dst_hbm.at[wid], dst_v, sem)]

        for k in range(nk):
            dsl[pl.ds(k * _L, _L)] = jnp.zeros((_L,), jnp.float32)
        pltpu.sync_copy(dsl, sh_deg.at[pl.ds(base, sl)])
        pltpu.sync_copy(dsl, sh_s.at[pl.ds(base, sl)])
        for k in range(_B // _L):
            valr[pl.ds(k * _L, _L)] = jnp.ones((_L,), jnp.float32)
        for c in cin:
            c.wait()
        plsc.subcore_barrier()

        _fire_drain(lambda j: pltpu.async_copy(
            valr, sh_deg.at[dst_v.at[j]], sem, add=True))
        plsc.subcore_barrier()

        pltpu.sync_copy(sh_deg.at[pl.ds(base, sl)], dsl)
        for k in range(nk):
            v = dsl[pl.ds(k * _L, _L)] + 1.0
            dsl[pl.ds(k * _L, _L)] = _rsqrt16(v)
        pltpu.sync_copy(dsl, sh_d.at[pl.ds(base, sl)])
        plsc.subcore_barrier()

        _fire_drain(lambda j: pltpu.async_copy(
            sh_d.at[dst_v.at[j]], vals.at[j], sem))
        _fire_drain(lambda j: pltpu.async_copy(
            vals.at[j], sh_s.at[src_v.at[j]], sem, add=True))
        plsc.subcore_barrier()

        pltpu.sync_copy(sh_s.at[pl.ds(base, sl)], ssl)
        for k in range(nk):
            d = dsl[pl.ds(k * _L, _L)]
            s = ssl[pl.ds(k * _L, _L)]
            ssl[pl.ds(k * _L, _L)] = d * s + d * d
        pltpu.sync_copy(ssl, w_hbm.at[pl.ds(base, sl)])

    return sc_w


def _tc_head(nfloat):
    def body(w_ref, x_ref, wg_ref, bg_ref, w1_ref, b1_ref, w2_ref, b2_ref,
             wc_ref, bc_ref, out_ref):
        xw = jnp.dot(w_ref[...], x_ref[...],
                     preferred_element_type=jnp.float32)
        pooled = jnp.dot(xw, wg_ref[...],
                         preferred_element_type=jnp.float32)
        pooled = pooled + nfloat * bg_ref[...]
        h = jnp.maximum(jnp.dot(pooled, w1_ref[...],
                                preferred_element_type=jnp.float32)
                        + b1_ref[...], 0.0)
        h = jnp.maximum(jnp.dot(h, w2_ref[...],
                                preferred_element_type=jnp.float32)
                        + b2_ref[...], 0.0)
        avg = jnp.dot(h, wc_ref[...],
                      preferred_element_type=jnp.float32) + bc_ref[...]
        m = jnp.max(avg, axis=1, keepdims=True)
        e = jnp.exp(avg - m)
        out_ref[...] = e / jnp.sum(e, axis=1, keepdims=True)
    return body


@jax.jit
def kernel(x, edge_index, W_gcn, b_gcn, W1, b1, W2, b2, Wc, bc):
    n, _ = x.shape
    e = edge_index.shape[1]

    rows = -(-e // (_NS * _B))
    epad = _NS * rows * _B
    npad = -(-(n + 1) // (_NS * _L)) * (_NS * _L)

    fill = jnp.full((epad - e,), n, jnp.int32)
    src3 = jnp.concatenate([edge_index[0], fill]).reshape(_NS, rows, _B)
    dst3 = jnp.concatenate([edge_index[1], fill]).reshape(_NS, rows, _B)

    w_full = _make_sc_w(rows, npad)(src3, dst3)
    w_row = w_full[:n].reshape(1, n)

    result = pl.pallas_call(
        _tc_head(float(n)),
        out_shape=jax.ShapeDtypeStruct((1, 2), jnp.float32),
    )(w_row, x, W_gcn, b_gcn.reshape(1, -1), W1, b1.reshape(1, -1),
      W2, b2.reshape(1, -1), Wc, bc.reshape(1, -1))
    return (result, x)


# speedup vs baseline: 92.3803x; 1.0635x over previous
import functools

import jax
import jax.numpy as jnp
from jax import lax
from jax.experimental import pallas as pl
from jax.experimental.pallas import tpu as pltpu
from jax.experimental.pallas import tpu_sc as plsc

_NS = 16
_B = 128
_L = 16


def _rsqrt16(v):
    t = 0.5 * (v + 1.0)
    for _ in range(13):
        t = 0.5 * (t + v / t)
    return 1.0 / t


def _make_sc_w(rows, npad, n):
    sl = npad // _NS
    nk = sl // _L
    tail = n - (_NS - 1) * sl
    mesh = plsc.VectorSubcoreMesh(
        core_axis_name="c", subcore_axis_name="s", num_cores=1)

    grp = 26

    def _fire_drain(mk):
        bounds = list(range(0, rows, grp)) + [rows]
        prev = [mk(j) for j in range(bounds[0], bounds[1])]
        for gi in range(1, len(bounds) - 1):
            cur = [mk(j) for j in range(bounds[gi], bounds[gi + 1])]
            for c in prev:
                c.wait()
            prev = cur
        for c in prev:
            c.wait()

    @functools.partial(
        pl.kernel,
        mesh=mesh,
        out_type=jax.ShapeDtypeStruct((n,), jnp.float32),
        scratch_types=[
            pltpu.VMEM((rows, _B), jnp.int32),
            pltpu.VMEM((rows, _B), jnp.int32),
            pltpu.VMEM((_B,), jnp.float32),
            pltpu.VMEM((rows, _B), jnp.float32),
            pltpu.VMEM((sl,), jnp.float32),
            pltpu.VMEM((sl,), jnp.float32),
            pltpu.VMEM_SHARED((npad,), jnp.float32),
            pltpu.VMEM_SHARED((npad,), jnp.float32),
            pltpu.VMEM_SHARED((npad,), jnp.float32),
            pltpu.SemaphoreType.DMA,
            pltpu.SemaphoreType.DMA,
            pltpu.SemaphoreType.DMA,
        ],
    )
    def sc_w(edges_hbm, w_hbm, src_v, dst_v, valr, vals, dsl, ssl,
             sh_deg, sh_d, sh_s, sem, sem2, sem3):
        wid = lax.axis_index("s")
        base = wid * sl

        cin = [pltpu.async_copy(edges_hbm.at[0, wid], src_v, sem),
               pltpu.async_copy(edges_hbm.at[1, wid], dst_v, sem)]

        for k in range(nk):
            dsl[pl.ds(k * _L, _L)] = jnp.zeros((_L,), jnp.float32)
        pltpu.sync_copy(dsl, sh_deg.at[pl.ds(base, sl)])
        pltpu.sync_copy(dsl, sh_s.at[pl.ds(base, sl)])
        for k in range(_B // _L):
            valr[pl.ds(k * _L, _L)] = jnp.ones((_L,), jnp.float32)
        for c in cin:
            c.wait()
        plsc.subcore_barrier()

        _fire_drain(lambda j: pltpu.async_copy(
            valr, sh_deg.at[dst_v.at[j]], sem, add=True))
        plsc.subcore_barrier()

        pltpu.sync_copy(sh_deg.at[pl.ds(base, sl)], dsl)
        for k in range(nk):
            v = dsl[pl.ds(k * _L, _L)] + 1.0
            dsl[pl.ds(k * _L, _L)] = _rsqrt16(v)
        pltpu.sync_copy(dsl, sh_d.at[pl.ds(base, sl)])
        plsc.subcore_barrier()

        bounds = list(range(0, rows, grp)) + [rows]
        groups = list(zip(bounds[:-1], bounds[1:]))
        gsems = [sem2 if gi % 2 == 0 else sem3 for gi in range(len(groups))]

        def fire_gather(gi):
            return [pltpu.async_copy(sh_d.at[dst_v.at[j]], vals.at[j],
                                     gsems[gi])
                    for j in range(*groups[gi])]

        gds = [fire_gather(0)]
        if len(groups) > 1:
            gds.append(fire_gather(1))
        scds = []
        for gi, (lo, hi) in enumerate(groups):
            for c in gds[gi]:
                c.wait()
            if gi + 2 < len(groups):
                gds.append(fire_gather(gi + 2))
            scds += [pltpu.async_copy(
                vals.at[j], sh_s.at[src_v.at[j]], sem, add=True)
                for j in range(lo, hi)]
        for c in scds:
            c.wait()
        plsc.subcore_barrier()

        pltpu.sync_copy(sh_s.at[pl.ds(base, sl)], ssl)
        for k in range(nk):
            d = dsl[pl.ds(k * _L, _L)]
            s = ssl[pl.ds(k * _L, _L)]
            ssl[pl.ds(k * _L, _L)] = d * s + d * d

        @pl.when(wid < _NS - 1)
        def _():
            pltpu.sync_copy(ssl, w_hbm.at[pl.ds(base, sl)])

        @pl.when(wid == _NS - 1)
        def _():
            pltpu.sync_copy(ssl.at[pl.ds(0, tail)],
                            w_hbm.at[pl.ds((_NS - 1) * sl, tail)])

    return sc_w


def _tc_head(nfloat):
    def body(w_ref, x_ref, wg_ref, bg_ref, w1_ref, b1_ref, w2_ref, b2_ref,
             wc_ref, bc_ref, out_ref):
        xw = jnp.dot(w_ref[...], x_ref[...],
                     preferred_element_type=jnp.float32)
        pooled = jnp.dot(xw, wg_ref[...],
                         preferred_element_type=jnp.float32)
        pooled = pooled + nfloat * bg_ref[...]
        h = jnp.maximum(jnp.dot(pooled, w1_ref[...],
                                preferred_element_type=jnp.float32)
                        + b1_ref[...], 0.0)
        h = jnp.maximum(jnp.dot(h, w2_ref[...],
                                preferred_element_type=jnp.float32)
                        + b2_ref[...], 0.0)
        avg = jnp.dot(h, wc_ref[...],
                      preferred_element_type=jnp.float32) + bc_ref[...]
        m = jnp.max(avg, axis=1, keepdims=True)
        e = jnp.exp(avg - m)
        out_ref[...] = e / jnp.sum(e, axis=1, keepdims=True)
    return body


@jax.jit
def kernel(x, edge_index, W_gcn, b_gcn, W1, b1, W2, b2, Wc, bc):
    n, _ = x.shape
    e = edge_index.shape[1]

    rows = -(-e // (_NS * _B))
    epad = _NS * rows * _B
    npad = -(-(n + 1) // (_NS * _L)) * (_NS * _L)

    edges = jnp.pad(edge_index, ((0, 0), (0, epad - e)),
                    constant_values=n).reshape(2, _NS, rows, _B)

    w_full = _make_sc_w(rows, npad, n)(edges)
    w_row = w_full.reshape(1, n)

    result = pl.pallas_call(
        _tc_head(float(n)),
        out_shape=jax.ShapeDtypeStruct((1, 2), jnp.float32),
    )(w_row, x, W_gcn, b_gcn.reshape(1, -1), W1, b1.reshape(1, -1),
      W2, b2.reshape(1, -1), Wc, bc.reshape(1, -1))
    return (result, x)
